# baseline (speedup 1.0000x reference)
import math

import jax
import jax.numpy as jnp
from jax import lax
from jax.experimental import pallas as pl
from jax.experimental.pallas import tpu as pltpu
from jax.experimental.pallas import tpu_sc as plsc

VOCAB = 1000000
D = 64
BATCH = 4096
HIST = 200

NC = 2
NS = 16
NW = NC * NS
L = 16

B = BATCH * HIST
B_PER_W = B // NW
CHUNK = 128
N_CHUNKS = B_PER_W // CHUNK

SCALE = math.sqrt(D)


def _body(idx_hbm, table_hbm, out_hbm, idx_v, rows_v, gsem):
    wid = lax.axis_index("s") * NC + lax.axis_index("c")
    pltpu.sync_copy(idx_hbm.at[wid], idx_v)

    @pl.loop(0, N_CHUNKS)
    def _chunk(t):
        pltpu.async_copy(table_hbm.at[idx_v.at[t]], rows_v, gsem).wait()

        @pl.loop(0, CHUNK)
        def _row(r):
            for j in range(D // L):
                sl = pl.ds(j * L, L)
                rows_v[r, sl] = rows_v[r, sl] * SCALE

        pltpu.sync_copy(rows_v, out_hbm.at[pl.ds(wid * B_PER_W + t * CHUNK, CHUNK)])


def kernel(x, table):
    idx = x.reshape(NW, N_CHUNKS, CHUNK).astype(jnp.int32)
    out = pl.kernel(
        _body,
        out_type=jax.ShapeDtypeStruct((B, D), jnp.float32),
        mesh=plsc.VectorSubcoreMesh(
            core_axis_name="c", subcore_axis_name="s", num_cores=NC, num_subcores=NS
        ),
        scratch_types=[
            pltpu.VMEM((N_CHUNKS, CHUNK), jnp.int32),
            pltpu.VMEM((CHUNK, D), jnp.float32),
            pltpu.SemaphoreType.DMA,
        ],
        compiler_params=pltpu.CompilerParams(use_tc_tiling_on_sc=False),
    )(idx, table)
    return out.reshape(BATCH, HIST, D)


# speedup vs baseline: 1.2086x; 1.2086x over previous
import math

import jax
import jax.numpy as jnp
from jax import lax
from jax.experimental import pallas as pl
from jax.experimental.pallas import tpu as pltpu
from jax.experimental.pallas import tpu_sc as plsc

VOCAB = 1000000
D = 64
BATCH = 4096
HIST = 200

NC = 2
NS = 16
NW = NC * NS
L = 16

B = BATCH * HIST
B_PER_W = B // NW
CHUNK = 128
N_CHUNKS = B_PER_W // CHUNK
K = 2
FILLS = N_CHUNKS // K
NBUF = 4

SCALE = math.sqrt(D)


def _body(idx_hbm, table_hbm, out_hbm, idx_v, rows_v, *sems):
    gsems, wsems = sems[:NBUF], sems[NBUF:]
    wid = lax.axis_index("s") * NC + lax.axis_index("c")
    pltpu.sync_copy(idx_hbm.at[wid], idx_v)

    def g_copy(b, f):
        return [
            pltpu.make_async_copy(
                table_hbm.at[idx_v.at[K * f + k]], rows_v.at[b, k], gsems[b]
            )
            for k in range(K)
        ]

    def w_copy(b, f):
        return [
            pltpu.make_async_copy(
                rows_v.at[b, k], out_hbm.at[wid * N_CHUNKS + K * f + k], wsems[b]
            )
            for k in range(K)
        ]

    def fire_gathers(b, f):
        for c in g_copy(b, f):
            c.start()

    def drain_gathers(b, f):
        for c in g_copy(b, f):
            c.wait()

    def fire_writes(b, f):
        for c in w_copy(b, f):
            c.start()

    def drain_writes(b, f):
        for c in w_copy(b, f):
            c.wait()

    def scale(b):
        @pl.loop(0, CHUNK)
        def _row(r):
            for k in range(K):
                for j in range(D // L):
                    sl = pl.ds(j * L, L)
                    rows_v[b, k, r, sl] = rows_v[b, k, r, sl] * SCALE

    fire_gathers(0, 0)
    fire_gathers(1, 1)

    @pl.loop(0, FILLS // NBUF)
    def _step(S):
        for q in range(NBUF):
            f = NBUF * S + q
            drain_gathers(q, f)
            scale(q)
            fire_writes(q, f)
            b2 = (q + 2) % NBUF
            if q >= 2:
                drain_writes(b2, f - 2)

                @pl.when(S < FILLS // NBUF - 1)
                def _():
                    fire_gathers(b2, f + 2)

            else:

                @pl.when(S >= 1)
                def _():
                    drain_writes(b2, f - 2)

                fire_gathers(b2, f + 2)

    drain_writes(2, FILLS - 2)
    drain_writes(3, FILLS - 1)


def kernel(x, table):
    idx = x.reshape(NW, N_CHUNKS, CHUNK).astype(jnp.int32)
    out = pl.kernel(
        _body,
        out_type=jax.ShapeDtypeStruct((NW * N_CHUNKS, CHUNK, D), jnp.float32),
        mesh=plsc.VectorSubcoreMesh(
            core_axis_name="c", subcore_axis_name="s", num_cores=NC, num_subcores=NS
        ),
        scratch_types=(
            [
                pltpu.VMEM((N_CHUNKS, CHUNK), jnp.int32),
                pltpu.VMEM((NBUF, K, CHUNK, D), jnp.float32),
            ]
            + [pltpu.SemaphoreType.DMA] * (2 * NBUF)
        ),
        compiler_params=pltpu.CompilerParams(use_tc_tiling_on_sc=False),
    )(idx, table)
    return out.reshape(BATCH, HIST, D)
